# baseline (speedup 1.0000x reference)
import functools

import jax
import jax.numpy as jnp
from jax import lax
from jax.experimental import pallas as pl
from jax.experimental.pallas import tpu as pltpu
from jax.experimental.pallas import tpu_sc as plsc

N = 10000
E = 320000
S = 128
TW = 256
XP = 16
NRBF = 18
RBP = 32
NC = 2
NS = 16
NW = NC * NS
EPW = E // NW
EPT = E // NS
GC = 80
NPT = N // NS

BE = 800
BN = 2000


def _gather_body(t_hbm, src_hbm, dst_hbm, ts_out, td_out,
                 srcv, dstv, tsv, tdv, sem):
    c = lax.axis_index("c")
    s = lax.axis_index("s")
    wid = s * NC + c
    base0 = wid * EPW

    def chunk(i, carry):
        base = base0 + i * GC
        pltpu.sync_copy(src_hbm.at[pl.ds(base, GC)], srcv)
        pltpu.sync_copy(dst_hbm.at[pl.ds(base, GC)], dstv)
        c1 = pltpu.async_copy(t_hbm.at[srcv], tsv, sem)
        c2 = pltpu.async_copy(t_hbm.at[dstv], tdv, sem)
        c1.wait()
        c2.wait()
        pltpu.sync_copy(tsv, ts_out.at[pl.ds(base, GC)])
        pltpu.sync_copy(tdv, td_out.at[pl.ds(base, GC)])
        return carry

    lax.fori_loop(0, EPW // GC, chunk, 0)


def _make_gather():
    mesh = plsc.VectorSubcoreMesh(core_axis_name="c", subcore_axis_name="s")
    return pl.kernel(
        _gather_body,
        out_type=(
            jax.ShapeDtypeStruct((E, TW), jnp.float32),
            jax.ShapeDtypeStruct((E, TW), jnp.float32),
        ),
        mesh=mesh,
        scratch_types=[
            pltpu.VMEM((GC,), jnp.int32),
            pltpu.VMEM((GC,), jnp.int32),
            pltpu.VMEM((GC, TW), jnp.float32),
            pltpu.VMEM((GC, TW), jnp.float32),
            pltpu.SemaphoreType.DMA,
        ],
    )


def _scatter_body(dh_hbm, dxp_hbm, dst_hbm, zeros_hbm,
                  aggh_out, aggx_out,
                  dstv, rowv, acc):
    c = lax.axis_index("c")
    s = lax.axis_index("s")
    base0 = s * EPT

    @pl.when(s == 0)
    def _():
        pltpu.sync_copy(zeros_hbm, acc)

    plsc.subcore_barrier()

    def make_chunk(msg_hbm):
        def chunk(i, carry):
            base = base0 + i * GC
            pltpu.sync_copy(dst_hbm.at[pl.ds(base, GC)], dstv)
            pltpu.sync_copy(msg_hbm.at[pl.ds(base, GC)], rowv)
            pltpu.sync_copy(rowv, acc.at[dstv], add=True)
            return carry
        return chunk

    @pl.when(c == 0)
    def _():
        lax.fori_loop(0, EPT // GC, make_chunk(dh_hbm), 0)

    @pl.when(c == 1)
    def _():
        lax.fori_loop(0, EPT // GC, make_chunk(dxp_hbm), 0)

    plsc.subcore_barrier()

    @pl.when((c == 0) & (s == 0))
    def _():
        pltpu.sync_copy(acc, aggh_out)

    @pl.when((c == 1) & (s == 0))
    def _():
        pltpu.sync_copy(acc, aggx_out)


def _make_scatter():
    mesh = plsc.VectorSubcoreMesh(core_axis_name="c", subcore_axis_name="s")
    return pl.kernel(
        _scatter_body,
        out_type=(
            jax.ShapeDtypeStruct((N, S), jnp.float32),
            jax.ShapeDtypeStruct((N, S), jnp.float32),
        ),
        mesh=mesh,
        scratch_types=[
            pltpu.VMEM((GC,), jnp.int32),
            pltpu.VMEM((GC, S), jnp.float32),
            pltpu.VMEM_SHARED((N, S), jnp.float32),
        ],
    )


def _silu(z):
    return z * jax.nn.sigmoid(z)


def _mlp_body(ts, td, es, cent, wiv,
              w1a, w1b, w1c, w1d, b1, w2, b2, w3, b3,
              wg1, bg1, wg2, bg2, wh, bh, wx1, bx1, wx2, bx2,
              dh_out, dx_out):
    hs = ts[:, :S]
    hd = td[:, :S]
    r = ts[:, S:] - td[:, S:]
    d2 = jnp.sum(r * r, axis=1, keepdims=True)
    dist = jnp.sqrt(d2)
    t = (dist - cent[...]) * wiv[0, 0]
    rbf = jnp.exp(-(t * t))

    bf = jnp.bfloat16

    def dot(a_, b_):
        return jnp.dot(a_.astype(bf), b_.astype(bf),
                       preferred_element_type=jnp.float32)

    m = (dot(hs, w1a[...]) + dot(hd, w1b[...])
         + dot(rbf, w1c[...]) + dot(es[...], w1d[...]) + b1[...])
    m = _silu(m)
    m = _silu(dot(m, w2[...]) + b2[...])
    m = _silu(dot(m, w3[...]) + b3[...])
    a = jax.nn.relu(dot(m, wg1[...]) + bg1[...])
    g = jax.nn.sigmoid(jnp.sum(a * wg2[...], axis=1, keepdims=True) + bg2[0, 0])
    m = m * g
    dh_out[...] = _silu(dot(m, wh[...]) + bh[...])
    cx = _silu(dot(m, wx1[...]) + bx1[...])
    coeff = (jnp.sum(cx * wx2[...], axis=1, keepdims=True) + bx2[0, 0]) * 0.08
    dx_out[...] = r * coeff


def _full(shape):
    return pl.BlockSpec(shape, lambda i: (0,) * len(shape))


def _mlp_call(ts, td, es, cent, wiv, wts):
    in_specs = [
        pl.BlockSpec((BE, TW), lambda i: (i, 0)),
        pl.BlockSpec((BE, TW), lambda i: (i, 0)),
        pl.BlockSpec((BE, 16), lambda i: (i, 0)),
        _full(cent.shape), _full(wiv.shape),
    ] + [_full(w.shape) for w in wts]
    return pl.pallas_call(
        _mlp_body,
        grid=(E // BE,),
        in_specs=in_specs,
        out_specs=[
            pl.BlockSpec((BE, S), lambda i: (i, 0)),
            pl.BlockSpec((BE, S), lambda i: (i, 0)),
        ],
        out_shape=[
            jax.ShapeDtypeStruct((E, S), jnp.float32),
            jax.ShapeDtypeStruct((E, S), jnp.float32),
        ],
    )(ts, td, es, cent, wiv, *wts)


def _fin_body(h, x16, aggh, aggx, lng, lnb, alpha, h_out, x_out):
    sa = jax.nn.sigmoid(alpha[0, 0])
    pre = h[...] + sa * aggh[...]
    mu = jnp.mean(pre, axis=1, keepdims=True)
    cent = pre - mu
    var = jnp.mean(cent * cent, axis=1, keepdims=True)
    h_out[...] = cent * lax.rsqrt(var + 1e-5) * lng[...] + lnb[...]
    x_out[...] = x16[...] + aggx[:, :XP]


def _fin_call(h, x16, aggh, aggx, lng, lnb, alpha):
    return pl.pallas_call(
        _fin_body,
        grid=(N // BN,),
        in_specs=[
            pl.BlockSpec((BN, S), lambda i: (i, 0)),
            pl.BlockSpec((BN, XP), lambda i: (i, 0)),
            pl.BlockSpec((BN, S), lambda i: (i, 0)),
            pl.BlockSpec((BN, S), lambda i: (i, 0)),
            _full(lng.shape), _full(lnb.shape), _full(alpha.shape),
        ],
        out_specs=[
            pl.BlockSpec((BN, S), lambda i: (i, 0)),
            pl.BlockSpec((BN, XP), lambda i: (i, 0)),
        ],
        out_shape=[
            jax.ShapeDtypeStruct((N, S), jnp.float32),
            jax.ShapeDtypeStruct((N, XP), jnp.float32),
        ],
    )(h, x16, aggh, aggx, lng, lnb, alpha)


def kernel(h, x, edge_index, e_s, params, centers, widths):
    src = edge_index[0].astype(jnp.int32)
    dst = edge_index[1].astype(jnp.int32)
    xf = x.astype(jnp.float32)
    x16 = jnp.pad(xf, ((0, 0), (0, XP - 3)))
    T = jnp.concatenate([h, xf, jnp.zeros((N, TW - S - 3), jnp.float32)], axis=1)

    W1 = params['W1']
    w1a = W1[:, :S].T
    w1b = W1[:, S:2 * S].T
    w1c = jnp.pad(W1[:, 2 * S:2 * S + NRBF].T, ((0, RBP - NRBF), (0, 0)))
    w1d = W1[:, 2 * S + NRBF:].T
    b1 = params['b1'][None, :]
    w2 = params['W2'].T
    b2 = params['b2'][None, :]
    w3 = params['W3'].T
    b3 = params['b3'][None, :]
    wg1 = params['Wg1'].T
    bg1 = params['bg1'][None, :]
    wg2 = params['Wg2']
    bg2 = params['bg2'][None, :]
    wh = params['Wh'].T
    bh = params['bh'][None, :]
    wx1 = params['Wx1'].T
    bx1 = params['bx1'][None, :]
    wx2 = params['Wx2']
    bx2 = params['bx2'][None, :]
    cent = jnp.pad(centers[None, :], ((0, 0), (0, RBP - NRBF)))
    wiv = (1.0 / (widths + 1e-8)).reshape(1, 1)
    alpha = params['alpha'].reshape(1, 1)
    lng = params['ln_g'][None, :]
    lnb = params['ln_b'][None, :]

    ts, td = _make_gather()(T, src, dst)

    wts = [w1a, w1b, w1c, w1d, b1, w2, b2, w3, b3,
           wg1, bg1, wg2, bg2, wh, bh, wx1, bx1, wx2, bx2]
    dh, dxp = _mlp_call(ts, td, e_s, cent, wiv, wts)

    zeros = jnp.zeros((N, S), jnp.float32)
    aggh, aggx = _make_scatter()(dh, dxp, dst, zeros)

    h_new, x_new16 = _fin_call(h, x16, aggh, aggx, lng, lnb, alpha)
    return (h_new, x_new16[:, :3])


# speedup vs baseline: 1.1776x; 1.1776x over previous
import functools

import jax
import jax.numpy as jnp
from jax import lax
from jax.experimental import pallas as pl
from jax.experimental.pallas import tpu as pltpu
from jax.experimental.pallas import tpu_sc as plsc

N = 10000
E = 320000
S = 128
TW = 256
XP = 16
NRBF = 18
RBP = 32
NC = 2
NS = 16
NW = NC * NS
NSL = 2

BE = 800
BN = 2000


def _chunk_of(n):
    for g in range(128, 0, -8):
        if n % g == 0:
            return g
    raise ValueError(n)


def _make_gather(esl):
    epw = esl // NW
    gc = _chunk_of(epw)

    def body(t_hbm, src_hbm, dst_hbm, ts_out, td_out,
             srcv, dstv, tsv, tdv, sem):
        c = lax.axis_index("c")
        s = lax.axis_index("s")
        wid = s * NC + c
        base0 = wid * epw

        def chunk(i, carry):
            base = base0 + i * gc
            pltpu.sync_copy(src_hbm.at[pl.ds(base, gc)], srcv)
            pltpu.sync_copy(dst_hbm.at[pl.ds(base, gc)], dstv)
            c1 = pltpu.async_copy(t_hbm.at[srcv], tsv, sem)
            c2 = pltpu.async_copy(t_hbm.at[dstv], tdv, sem)
            c1.wait()
            c2.wait()
            pltpu.sync_copy(tsv, ts_out.at[pl.ds(base, gc)])
            pltpu.sync_copy(tdv, td_out.at[pl.ds(base, gc)])
            return carry

        lax.fori_loop(0, epw // gc, chunk, 0)

    mesh = plsc.VectorSubcoreMesh(core_axis_name="c", subcore_axis_name="s")
    return pl.kernel(
        body,
        out_type=(
            jax.ShapeDtypeStruct((esl, TW), jnp.float32),
            jax.ShapeDtypeStruct((esl, TW), jnp.float32),
        ),
        mesh=mesh,
        scratch_types=[
            pltpu.VMEM((gc,), jnp.int32),
            pltpu.VMEM((gc,), jnp.int32),
            pltpu.VMEM((gc, TW), jnp.float32),
            pltpu.VMEM((gc, TW), jnp.float32),
            pltpu.SemaphoreType.DMA,
        ],
    )


def _make_scatter(esl):
    ept = esl // NS
    gc = _chunk_of(ept)

    def body(dh_hbm, dxp_hbm, dst_hbm, zeros_hbm,
             aggh_out, aggx_out,
             dstv, rowv, acc):
        c = lax.axis_index("c")
        s = lax.axis_index("s")
        base0 = s * ept

        @pl.when(s == 0)
        def _():
            pltpu.sync_copy(zeros_hbm, acc)

        plsc.subcore_barrier()

        def make_chunk(msg_hbm):
            def chunk(i, carry):
                base = base0 + i * gc
                pltpu.sync_copy(dst_hbm.at[pl.ds(base, gc)], dstv)
                pltpu.sync_copy(msg_hbm.at[pl.ds(base, gc)], rowv)
                pltpu.sync_copy(rowv, acc.at[dstv], add=True)
                return carry
            return chunk

        @pl.when(c == 0)
        def _():
            lax.fori_loop(0, ept // gc, make_chunk(dh_hbm), 0)

        @pl.when(c == 1)
        def _():
            lax.fori_loop(0, ept // gc, make_chunk(dxp_hbm), 0)

        plsc.subcore_barrier()

        @pl.when((c == 0) & (s == 0))
        def _():
            pltpu.sync_copy(acc, aggh_out)

        @pl.when((c == 1) & (s == 0))
        def _():
            pltpu.sync_copy(acc, aggx_out)

    mesh = plsc.VectorSubcoreMesh(core_axis_name="c", subcore_axis_name="s")
    return pl.kernel(
        body,
        out_type=(
            jax.ShapeDtypeStruct((N, S), jnp.float32),
            jax.ShapeDtypeStruct((N, S), jnp.float32),
        ),
        mesh=mesh,
        scratch_types=[
            pltpu.VMEM((gc,), jnp.int32),
            pltpu.VMEM((gc, S), jnp.float32),
            pltpu.VMEM_SHARED((N, S), jnp.float32),
        ],
    )


def _silu(z):
    return z * jax.nn.sigmoid(z)


def _mlp_body(ts, td, es, cent, wiv,
              w1a, w1b, w1c, w1d, b1, w2, b2, w3, b3,
              wg1, bg1, wg2, bg2, wh, bh, wx1, bx1, wx2, bx2,
              dh_out, dx_out):
    hs = ts[:, :S]
    hd = td[:, :S]
    r = ts[:, S:] - td[:, S:]
    d2 = jnp.sum(r * r, axis=1, keepdims=True)
    dist = jnp.sqrt(d2)
    t = (dist - cent[...]) * wiv[0, 0]
    rbf = jnp.exp(-(t * t))

    bf = jnp.bfloat16

    def dot(a_, b_):
        return jnp.dot(a_.astype(bf), b_.astype(bf),
                       preferred_element_type=jnp.float32)

    m = (dot(hs, w1a[...]) + dot(hd, w1b[...])
         + dot(rbf, w1c[...]) + dot(es[...], w1d[...]) + b1[...])
    m = _silu(m)
    m = _silu(dot(m, w2[...]) + b2[...])
    m = _silu(dot(m, w3[...]) + b3[...])
    a = jax.nn.relu(dot(m, wg1[...]) + bg1[...])
    g = jax.nn.sigmoid(jnp.sum(a * wg2[...], axis=1, keepdims=True) + bg2[0, 0])
    m = m * g
    dh_out[...] = _silu(dot(m, wh[...]) + bh[...])
    cx = _silu(dot(m, wx1[...]) + bx1[...])
    coeff = (jnp.sum(cx * wx2[...], axis=1, keepdims=True) + bx2[0, 0]) * 0.08
    dx_out[...] = r * coeff


def _full(shape):
    return pl.BlockSpec(shape, lambda i: (0,) * len(shape))


def _mlp_call(ts, td, es, cent, wiv, wts):
    esl = ts.shape[0]
    in_specs = [
        pl.BlockSpec((BE, TW), lambda i: (i, 0)),
        pl.BlockSpec((BE, TW), lambda i: (i, 0)),
        pl.BlockSpec((BE, 16), lambda i: (i, 0)),
        _full(cent.shape), _full(wiv.shape),
    ] + [_full(w.shape) for w in wts]
    return pl.pallas_call(
        _mlp_body,
        grid=(esl // BE,),
        in_specs=in_specs,
        out_specs=[
            pl.BlockSpec((BE, S), lambda i: (i, 0)),
            pl.BlockSpec((BE, S), lambda i: (i, 0)),
        ],
        out_shape=[
            jax.ShapeDtypeStruct((esl, S), jnp.float32),
            jax.ShapeDtypeStruct((esl, S), jnp.float32),
        ],
    )(ts, td, es, cent, wiv, *wts)


def _fin_body(h, x16, lng, lnb, alpha, *refs):
    n_agg = (len(refs) - 2) // 2
    agghs = refs[:n_agg]
    aggxs = refs[n_agg:2 * n_agg]
    h_out, x_out = refs[2 * n_agg:]
    sa = jax.nn.sigmoid(alpha[0, 0])
    aggh = agghs[0][...]
    aggx = aggxs[0][...]
    for k in range(1, n_agg):
        aggh = aggh + agghs[k][...]
        aggx = aggx + aggxs[k][...]
    pre = h[...] + sa * aggh
    mu = jnp.mean(pre, axis=1, keepdims=True)
    cent = pre - mu
    var = jnp.mean(cent * cent, axis=1, keepdims=True)
    h_out[...] = cent * lax.rsqrt(var + 1e-5) * lng[...] + lnb[...]
    x_out[...] = x16[...] + aggx[:, :XP]


def _fin_call(h, x16, lng, lnb, alpha, agghs, aggxs):
    nsb = pl.BlockSpec((BN, S), lambda i: (i, 0))
    return pl.pallas_call(
        _fin_body,
        grid=(N // BN,),
        in_specs=[
            nsb,
            pl.BlockSpec((BN, XP), lambda i: (i, 0)),
            _full(lng.shape), _full(lnb.shape), _full(alpha.shape),
        ] + [nsb] * (len(agghs) + len(aggxs)),
        out_specs=[
            nsb,
            pl.BlockSpec((BN, XP), lambda i: (i, 0)),
        ],
        out_shape=[
            jax.ShapeDtypeStruct((N, S), jnp.float32),
            jax.ShapeDtypeStruct((N, XP), jnp.float32),
        ],
    )(h, x16, lng, lnb, alpha, *agghs, *aggxs)


def kernel(h, x, edge_index, e_s, params, centers, widths):
    src = edge_index[0].astype(jnp.int32)
    dst = edge_index[1].astype(jnp.int32)
    xf = x.astype(jnp.float32)
    x16 = jnp.pad(xf, ((0, 0), (0, XP - 3)))
    T = jnp.concatenate([h, xf, jnp.zeros((N, TW - S - 3), jnp.float32)], axis=1)

    W1 = params['W1']
    w1a = W1[:, :S].T
    w1b = W1[:, S:2 * S].T
    w1c = jnp.pad(W1[:, 2 * S:2 * S + NRBF].T, ((0, RBP - NRBF), (0, 0)))
    w1d = W1[:, 2 * S + NRBF:].T
    b1 = params['b1'][None, :]
    w2 = params['W2'].T
    b2 = params['b2'][None, :]
    w3 = params['W3'].T
    b3 = params['b3'][None, :]
    wg1 = params['Wg1'].T
    bg1 = params['bg1'][None, :]
    wg2 = params['Wg2']
    bg2 = params['bg2'][None, :]
    wh = params['Wh'].T
    bh = params['bh'][None, :]
    wx1 = params['Wx1'].T
    bx1 = params['bx1'][None, :]
    wx2 = params['Wx2']
    bx2 = params['bx2'][None, :]
    cent = jnp.pad(centers[None, :], ((0, 0), (0, RBP - NRBF)))
    wiv = (1.0 / (widths + 1e-8)).reshape(1, 1)
    alpha = params['alpha'].reshape(1, 1)
    lng = params['ln_g'][None, :]
    lnb = params['ln_b'][None, :]

    wts = [w1a, w1b, w1c, w1d, b1, w2, b2, w3, b3,
           wg1, bg1, wg2, bg2, wh, bh, wx1, bx1, wx2, bx2]
    zeros = jnp.zeros((N, S), jnp.float32)

    esl = E // NSL
    gather_fn = _make_gather(esl)
    scatter_fn = _make_scatter(esl)
    agghs, aggxs = [], []
    for k in range(NSL):
        sl = slice(k * esl, (k + 1) * esl)
        ts, td = gather_fn(T, src[sl], dst[sl])
        dh, dxp = _mlp_call(ts, td, e_s[sl], cent, wiv, wts)
        aggh_k, aggx_k = scatter_fn(dh, dxp, dst[sl], zeros)
        agghs.append(aggh_k)
        aggxs.append(aggx_k)

    h_new, x_new16 = _fin_call(h, x16, lng, lnb, alpha, agghs, aggxs)
    return (h_new, x_new16[:, :3])
